# baseline (speedup 1.0000x reference)
import functools
import jax
import jax.numpy as jnp
from jax import lax
from jax.experimental import pallas as pl
from jax.experimental.pallas import tpu as pltpu
from jax.experimental.pallas import tpu_sc as plsc


def _make_sc_add(B, S, D, CH=16):
    info = plsc.get_sparse_core_info()
    NC = info.num_cores
    NW = NC * info.num_subcores
    RW = S // NW
    NCH = RW // CH
    NV = D // info.num_lanes
    mesh = plsc.VectorSubcoreMesh(core_axis_name="c", subcore_axis_name="s")

    def body(x_hbm, e_hbm, o_hbm,
             ev0, ev1, xv0, xv1, ov0, ov1,
             se0, se1, sx0, sx1, so0, so1):
        wid = lax.axis_index("s") * NC + lax.axis_index("c")
        base = wid * RW
        evs, ses = (ev0, ev1), (se0, se1)
        xvs, sxs = (xv0, xv1), (sx0, sx1)
        ovs, sos = (ov0, ov1), (so0, so1)

        def e_src(c):
            return e_hbm.at[pl.ds(base + c * CH, CH)]

        def x_src(step):
            c, b = divmod(step, B)
            return x_hbm.at[b, pl.ds(base + c * CH, CH)]

        def o_dst(step):
            c, b = divmod(step, B)
            return o_hbm.at[b, pl.ds(base + c * CH, CH)]

        NSTEP = NCH * B
        pltpu.async_copy(e_src(0), ev0, se0)
        pltpu.async_copy(x_src(0), xv0, sx0)
        for step in range(NSTEP):
            c, b = divmod(step, B)
            if b == 0:
                if c + 1 < NCH:
                    pltpu.async_copy(e_src(c + 1), evs[(c + 1) % 2],
                                     ses[(c + 1) % 2])
                pltpu.make_async_copy(e_src(c), evs[c % 2], ses[c % 2]).wait()
            if step + 1 < NSTEP:
                pltpu.async_copy(x_src(step + 1), xvs[(step + 1) % 2],
                                 sxs[(step + 1) % 2])
            pltpu.make_async_copy(x_src(step), xvs[step % 2],
                                  sxs[step % 2]).wait()
            if step >= 2:
                pltpu.make_async_copy(ovs[step % 2], o_dst(step - 2),
                                      sos[step % 2]).wait()
            xv, ev, ov = xvs[step % 2], evs[c % 2], ovs[step % 2]

            def cbody(i, _):
                r = i // NV
                col = (i % NV) * 16
                ov[r, pl.ds(col, 16)] = (xv[r, pl.ds(col, 16)]
                                         + ev[r, pl.ds(col, 16)])
                return 0

            lax.fori_loop(0, CH * NV, cbody, 0)
            pltpu.async_copy(ov, o_dst(step), sos[step % 2])
        pltpu.make_async_copy(ovs[(NSTEP - 2) % 2], o_dst(NSTEP - 2),
                              sos[(NSTEP - 2) % 2]).wait()
        pltpu.make_async_copy(ovs[(NSTEP - 1) % 2], o_dst(NSTEP - 1),
                              sos[(NSTEP - 1) % 2]).wait()

    vm = lambda: pltpu.VMEM((CH, D), jnp.float32)
    return pl.kernel(
        body,
        mesh=mesh,
        out_type=jax.ShapeDtypeStruct((B, S, D), jnp.float32),
        scratch_types=[vm(), vm(), vm(), vm(), vm(), vm()]
        + [pltpu.SemaphoreType.DMA] * 6,
    )


def kernel(inputs, embeddings):
    B, S, D = inputs.shape
    return _make_sc_add(B, S, D)(inputs, embeddings[:S])


# speedup vs baseline: 2.2164x; 2.2164x over previous
import functools
import jax
import jax.numpy as jnp
from jax import lax
from jax.experimental import pallas as pl
from jax.experimental.pallas import tpu as pltpu
from jax.experimental.pallas import tpu_sc as plsc


def _make_sc_add(B, S, D, CH=16):
    info = plsc.get_sparse_core_info()
    NC = info.num_cores
    NW = NC * info.num_subcores
    RW = S // NW
    NCH = RW // CH
    NV = D // info.num_lanes
    mesh = plsc.VectorSubcoreMesh(core_axis_name="c", subcore_axis_name="s")

    def body(x_hbm, e_hbm, o_hbm,
             ev0, ev1, xv0, xv1, xv2, xv3,
             se0, se1, sx0, sx1, sx2, sx3, so0, so1, so2, so3):
        wid = lax.axis_index("s") * NC + lax.axis_index("c")
        base = wid * RW
        evs, ses = (ev0, ev1), (se0, se1)
        xvs = (xv0, xv1, xv2, xv3)
        sxs = (sx0, sx1, sx2, sx3)
        sos = (so0, so1, so2, so3)

        def e_src(c):
            return e_hbm.at[pl.ds(base + c * CH, CH)]

        def x_src(step):
            c, b = divmod(step, B)
            return x_hbm.at[b, pl.ds(base + c * CH, CH)]

        def o_dst(step):
            c, b = divmod(step, B)
            return o_hbm.at[b, pl.ds(base + c * CH, CH)]

        NSTEP = NCH * B
        pltpu.async_copy(e_src(0), ev0, se0)
        pltpu.async_copy(x_src(0), xv0, sx0)
        pltpu.async_copy(x_src(1), xv1, sx1)
        for step in range(NSTEP):
            c, b = divmod(step, B)
            if b == 0:
                if c + 1 < NCH:
                    pltpu.async_copy(e_src(c + 1), evs[(c + 1) % 2],
                                     ses[(c + 1) % 2])
                pltpu.make_async_copy(e_src(c), evs[c % 2], ses[c % 2]).wait()
            if step >= 2:
                pltpu.make_async_copy(xvs[(step - 2) % 4], o_dst(step - 2),
                                      sos[(step - 2) % 4]).wait()
            if step + 2 < NSTEP:
                pltpu.async_copy(x_src(step + 2), xvs[(step + 2) % 4],
                                 sxs[(step + 2) % 4])
            pltpu.make_async_copy(x_src(step), xvs[step % 4],
                                  sxs[step % 4]).wait()
            xv, ev = xvs[step % 4], evs[c % 2]

            @plsc.parallel_loop(0, CH * NV, 1, unroll=8)
            def cbody(i):
                r = i // NV
                col = (i % NV) * 16
                plsc.addupdate(xv.at[r, pl.ds(col, 16)],
                               ev[r, pl.ds(col, 16)])

            pltpu.async_copy(xv, o_dst(step), sos[step % 4])
        pltpu.make_async_copy(xvs[(NSTEP - 2) % 4], o_dst(NSTEP - 2),
                              sos[(NSTEP - 2) % 4]).wait()
        pltpu.make_async_copy(xvs[(NSTEP - 1) % 4], o_dst(NSTEP - 1),
                              sos[(NSTEP - 1) % 4]).wait()

    vm = lambda: pltpu.VMEM((CH, D), jnp.float32)
    return pl.kernel(
        body,
        mesh=mesh,
        out_type=jax.ShapeDtypeStruct((B, S, D), jnp.float32),
        scratch_types=[vm(), vm(), vm(), vm(), vm(), vm()]
        + [pltpu.SemaphoreType.DMA] * 10,
    )


def kernel(inputs, embeddings):
    B, S, D = inputs.shape
    return _make_sc_add(B, S, D)(inputs, embeddings[:S])


# speedup vs baseline: 3.1042x; 1.4005x over previous
import functools
import jax
import jax.numpy as jnp
from jax import lax
from jax.experimental import pallas as pl
from jax.experimental.pallas import tpu as pltpu
from jax.experimental.pallas import tpu_sc as plsc


def _tc_add_body(x_ref, e_ref, o_ref):
    o_ref[...] = x_ref[...] + e_ref[...]


def _tc_add(inputs, pos, SBLK=512):
    B, S, D = inputs.shape
    n_sblk = S // SBLK
    return pl.pallas_call(
        _tc_add_body,
        grid=(n_sblk,),
        in_specs=[
            pl.BlockSpec((B, SBLK, D), lambda s: (0, s, 0)),
            pl.BlockSpec((SBLK, D), lambda s: (s, 0)),
        ],
        out_specs=pl.BlockSpec((B, SBLK, D), lambda s: (0, s, 0)),
        out_shape=jax.ShapeDtypeStruct((B, S, D), inputs.dtype),
    )(inputs, pos)


def _make_sc_add(B, S, D, CH=16):
    info = plsc.get_sparse_core_info()
    NC = info.num_cores
    NW = NC * info.num_subcores
    RW = S // NW
    NCH = RW // CH
    NV = D // info.num_lanes
    mesh = plsc.VectorSubcoreMesh(core_axis_name="c", subcore_axis_name="s")

    def body(x_hbm, e_hbm, o_hbm,
             ev0, ev1, xv0, xv1, xv2, xv3,
             se0, se1, sx0, sx1, sx2, sx3, so0, so1, so2, so3):
        wid = lax.axis_index("s") * NC + lax.axis_index("c")
        base = wid * RW
        evs, ses = (ev0, ev1), (se0, se1)
        xvs = (xv0, xv1, xv2, xv3)
        sxs = (sx0, sx1, sx2, sx3)
        sos = (so0, so1, so2, so3)

        def e_src(c):
            return e_hbm.at[pl.ds(base + c * CH, CH)]

        def x_src(step):
            c, b = divmod(step, B)
            return x_hbm.at[b, pl.ds(base + c * CH, CH)]

        def o_dst(step):
            c, b = divmod(step, B)
            return o_hbm.at[b, pl.ds(base + c * CH, CH)]

        NSTEP = NCH * B
        pltpu.async_copy(e_src(0), ev0, se0)
        pltpu.async_copy(x_src(0), xv0, sx0)
        pltpu.async_copy(x_src(1), xv1, sx1)
        for step in range(NSTEP):
            c, b = divmod(step, B)
            if b == 0:
                if c + 1 < NCH:
                    pltpu.async_copy(e_src(c + 1), evs[(c + 1) % 2],
                                     ses[(c + 1) % 2])
                pltpu.make_async_copy(e_src(c), evs[c % 2], ses[c % 2]).wait()
            if step >= 2:
                pltpu.make_async_copy(xvs[(step - 2) % 4], o_dst(step - 2),
                                      sos[(step - 2) % 4]).wait()
            if step + 2 < NSTEP:
                pltpu.async_copy(x_src(step + 2), xvs[(step + 2) % 4],
                                 sxs[(step + 2) % 4])
            pltpu.make_async_copy(x_src(step), xvs[step % 4],
                                  sxs[step % 4]).wait()
            xv, ev = xvs[step % 4], evs[c % 2]

            @plsc.parallel_loop(0, CH * NV, 1, unroll=8)
            def cbody(i):
                r = i // NV
                col = (i % NV) * 16
                plsc.addupdate(xv.at[r, pl.ds(col, 16)],
                               ev[r, pl.ds(col, 16)])

            pltpu.async_copy(xv, o_dst(step), sos[step % 4])
        pltpu.make_async_copy(xvs[(NSTEP - 2) % 4], o_dst(NSTEP - 2),
                              sos[(NSTEP - 2) % 4]).wait()
        pltpu.make_async_copy(xvs[(NSTEP - 1) % 4], o_dst(NSTEP - 1),
                              sos[(NSTEP - 1) % 4]).wait()

    vm = lambda: pltpu.VMEM((CH, D), jnp.float32)
    return pl.kernel(
        body,
        mesh=mesh,
        out_type=jax.ShapeDtypeStruct((B, S, D), jnp.float32),
        scratch_types=[vm(), vm(), vm(), vm(), vm(), vm()]
        + [pltpu.SemaphoreType.DMA] * 10,
    )


def kernel(inputs, embeddings):
    B, S, D = inputs.shape
    return _tc_add(inputs, embeddings[:S])
